# baseline (speedup 1.0000x reference)
import functools

import jax
import jax.numpy as jnp
from jax import lax
from jax.experimental import pallas as pl
from jax.experimental.pallas import tpu as pltpu
from jax.experimental.pallas import tpu_sc as plsc

NUM_CORES = 2
NUM_SUBCORES = 16
NUM_WORKERS = NUM_CORES * NUM_SUBCORES
LANES = 16
BBLK = 128


def _gather_kernel(b0, b1, d):
    njd = d // 8
    ngrp = BBLK // LANES
    mesh = plsc.VectorSubcoreMesh(core_axis_name="c", subcore_axis_name="s")

    @functools.partial(
        pl.kernel,
        mesh=mesh,
        out_type=jax.ShapeDtypeStruct((b1, njd, b0 // BBLK, 8, BBLK),
                                      jnp.float32),
        scratch_types=[
            pltpu.VMEM((BBLK,), jnp.int32),
            pltpu.VMEM((BBLK,), jnp.int32),
            pltpu.VMEM((BBLK,), jnp.int32),
            pltpu.VMEM((BBLK, 2 * d), jnp.float32),
            pltpu.VMEM((BBLK, 2 * d), jnp.float32),
            pltpu.VMEM((njd, 8, BBLK), jnp.float32),
            pltpu.SemaphoreType.DMA,
            pltpu.SemaphoreType.DMA,
        ],
        compiler_params=pltpu.CompilerParams(
            use_tc_tiling_on_sc=True, needs_layout_passes=False),
    )
    def k(xt_hbm, tab_hbm, out_hbm, idx_v, m_v0, m_v1, rows_v0, rows_v1,
          outt_v, gsem0, gsem1):
        wid = lax.axis_index("s") * NUM_CORES + lax.axis_index("c")
        base = wid * BBLK

        def stage(s, m_v):
            pltpu.sync_copy(xt_hbm.at[s, pl.ds(base, BBLK)], idx_v)
            hs = []
            for g in range(ngrp):
                v = idx_v[pl.ds(g * LANES, LANES)]
                m_v[pl.ds(g * LANES, LANES)] = lax.shift_right_logical(v, 1)
                hs.append((v & 1) * d)
            return hs

        def drain_transpose(rv, sem, h_cur):
            pltpu.make_async_copy(tab_hbm.at[m_v0], rv, sem).wait()
            biota = lax.iota(jnp.int32, LANES)
            for g in range(ngrp):
                bidx = biota + g * LANES
                col0 = h_cur[g]

                def dcol_body(jd, c):
                    for d8 in range(8):
                        vals = plsc.load_gather(
                            rv, [bidx, col0 + (jd * 8 + d8)])
                        outt_v[jd, d8, pl.ds(g * LANES, LANES)] = vals
                    return c

                lax.fori_loop(0, njd, dcol_body, 0)

        h_first = stage(0, m_v0)
        pltpu.async_copy(tab_hbm.at[m_v0], rows_v0, gsem0)

        def full_body(s, h_cur):
            p_is0 = lax.rem(s, 2) == 0
            more = s + 1 < b1

            @pl.when(jnp.logical_and(more, p_is0))
            def _():
                stage(s + 1, m_v1)

            @pl.when(jnp.logical_and(more, jnp.logical_not(p_is0)))
            def _():
                stage(s + 1, m_v0)

            h_next = []
            for g in range(ngrp):
                v = idx_v[pl.ds(g * LANES, LANES)]
                h_next.append((v & 1) * d)

            @pl.when(jnp.logical_and(more, p_is0))
            def _():
                pltpu.async_copy(tab_hbm.at[m_v1], rows_v1, gsem1)

            @pl.when(jnp.logical_and(more, jnp.logical_not(p_is0)))
            def _():
                pltpu.async_copy(tab_hbm.at[m_v0], rows_v0, gsem0)

            @pl.when(p_is0)
            def _():
                drain_transpose(rows_v0, gsem0, h_cur)

            @pl.when(jnp.logical_not(p_is0))
            def _():
                drain_transpose(rows_v1, gsem1, h_cur)

            def wr(jd, c):
                pltpu.sync_copy(outt_v.at[jd], out_hbm.at[s, jd, wid])
                return c

            lax.fori_loop(0, njd, wr, 0)
            return h_next

        lax.fori_loop(0, b1, full_body, h_first)

    return k


def kernel(x, table):
    b0, b1 = x.shape
    v, d = table.shape
    xt = x.T
    tab = table.reshape(v // 2, 2 * d)
    out5 = _gather_kernel(b0, b1, d)(xt, tab)
    out = out5.transpose(2, 4, 0, 1, 3).reshape(b0, b1, d)
    return out


# speedup vs baseline: 1.9935x; 1.9935x over previous
import functools

import jax
import jax.numpy as jnp
from jax import lax
from jax.experimental import pallas as pl
from jax.experimental.pallas import tpu as pltpu
from jax.experimental.pallas import tpu_sc as plsc

NUM_CORES = 2
NUM_SUBCORES = 16
NUM_WORKERS = NUM_CORES * NUM_SUBCORES
BBLK = 128
PADW = 128


def _gather_kernel(b0, b1, d):
    mesh = plsc.VectorSubcoreMesh(core_axis_name="c", subcore_axis_name="s")

    @functools.partial(
        pl.kernel,
        mesh=mesh,
        out_type=jax.ShapeDtypeStruct((b0, b1, PADW), jnp.float32),
        scratch_types=[
            pltpu.VMEM((BBLK,), jnp.int32),
            pltpu.VMEM((BBLK,), jnp.int32),
            pltpu.VMEM((BBLK, PADW), jnp.float32),
            pltpu.VMEM((BBLK, PADW), jnp.float32),
            pltpu.SemaphoreType.DMA,
            pltpu.SemaphoreType.DMA,
            pltpu.SemaphoreType.DMA,
            pltpu.SemaphoreType.DMA,
        ],
        compiler_params=pltpu.CompilerParams(
            use_tc_tiling_on_sc=True, needs_layout_passes=False),
    )
    def k(xt_hbm, tab_hbm, out_hbm, idx_v0, idx_v1, rows_v0, rows_v1,
          gsem0, gsem1, wsem0, wsem1):
        wid = lax.axis_index("s") * NUM_CORES + lax.axis_index("c")
        base = wid * BBLK
        idx_v = (idx_v0, idx_v1)
        rows_v = (rows_v0, rows_v1)
        gsem = (gsem0, gsem1)
        wsem = (wsem0, wsem1)

        def out_slot(s):
            return out_hbm.at[pl.ds(base, BBLK), s]

        pltpu.sync_copy(xt_hbm.at[0, pl.ds(base, BBLK)], idx_v[0])
        pltpu.async_copy(tab_hbm.at[idx_v[0]], rows_v[0], gsem[0])

        def step(s, p):
            q = 1 - p
            @pl.when(s + 1 < b1)
            def _():
                @pl.when(s >= 1)
                def _():
                    pltpu.make_async_copy(
                        rows_v[q], out_slot(s + 1), wsem[q]).wait()
                pltpu.sync_copy(xt_hbm.at[s + 1, pl.ds(base, BBLK)], idx_v[q])
                pltpu.async_copy(tab_hbm.at[idx_v[q]], rows_v[q], gsem[q])

            pltpu.make_async_copy(tab_hbm.at[idx_v[p]], rows_v[p],
                                  gsem[p]).wait()
            pltpu.async_copy(rows_v[p], out_slot(s), wsem[p])

        def body(si, carry):
            s = si * 2
            step(s, 0)
            step(s + 1, 1)
            return carry

        lax.fori_loop(0, b1 // 2, body, 0)
        pltpu.make_async_copy(rows_v0, out_slot(0), wsem0).wait()
        pltpu.make_async_copy(rows_v1, out_slot(1), wsem1).wait()

    return k


def kernel(x, table):
    b0, b1 = x.shape
    v, d = table.shape
    xt = x.T
    tabp = jnp.pad(table, ((0, 0), (0, PADW - d)))
    outp = _gather_kernel(b0, b1, d)(xt, tabp)
    return outp[:, :, :d]


# speedup vs baseline: 2.0522x; 1.0294x over previous
import functools

import jax
import jax.numpy as jnp
from jax import lax
from jax.experimental import pallas as pl
from jax.experimental.pallas import tpu as pltpu
from jax.experimental.pallas import tpu_sc as plsc

NUM_CORES = 2
NUM_SUBCORES = 16
NUM_WORKERS = NUM_CORES * NUM_SUBCORES
BBLK = 128
PADW = 128
NBUF = 4


def _gather_kernel(b0, b1, d):
    mesh = plsc.VectorSubcoreMesh(core_axis_name="c", subcore_axis_name="s")
    assert b1 % NBUF == 0

    @functools.partial(
        pl.kernel,
        mesh=mesh,
        out_type=jax.ShapeDtypeStruct((b0, b1, PADW), jnp.float32),
        scratch_types=(
            [pltpu.VMEM((BBLK,), jnp.int32) for _ in range(NBUF)]
            + [pltpu.VMEM((BBLK, PADW), jnp.float32) for _ in range(NBUF)]
            + [pltpu.SemaphoreType.DMA for _ in range(2 * NBUF)]
        ),
        compiler_params=pltpu.CompilerParams(
            use_tc_tiling_on_sc=True, needs_layout_passes=False),
    )
    def k(xt_hbm, tab_hbm, out_hbm, *bufs):
        idx_v = bufs[0:NBUF]
        rows_v = bufs[NBUF:2 * NBUF]
        gsem = bufs[2 * NBUF:3 * NBUF]
        wsem = bufs[3 * NBUF:4 * NBUF]
        wid = lax.axis_index("s") * NUM_CORES + lax.axis_index("c")
        base = wid * BBLK

        def out_slot(s):
            return out_hbm.at[pl.ds(base, BBLK), s]

        def stage_fire(s, p):
            pltpu.sync_copy(xt_hbm.at[s, pl.ds(base, BBLK)], idx_v[p])
            pltpu.async_copy(tab_hbm.at[idx_v[p]], rows_v[p], gsem[p])

        for s in range(NBUF - 1):
            stage_fire(s, s)

        def body(si, carry):
            s0 = si * NBUF
            for k_ in range(NBUF):
                s = s0 + k_
                p = k_
                q = (k_ + NBUF - 1) % NBUF
                @pl.when(s + NBUF - 1 < b1)
                def _():
                    @pl.when(s >= 1)
                    def _():
                        pltpu.make_async_copy(
                            rows_v[q], out_slot(s), wsem[q]).wait()
                    stage_fire(s + NBUF - 1, q)

                pltpu.make_async_copy(tab_hbm.at[idx_v[p]], rows_v[p],
                                      gsem[p]).wait()
                pltpu.async_copy(rows_v[p], out_slot(s), wsem[p])
            return carry

        lax.fori_loop(0, b1 // NBUF, body, 0)
        for p in range(NBUF):
            pltpu.make_async_copy(rows_v[p], out_slot(p), wsem[p]).wait()

    return k


def kernel(x, table):
    b0, b1 = x.shape
    v, d = table.shape
    xt = x.T
    tabp = jnp.pad(table, ((0, 0), (0, PADW - d)))
    outp = _gather_kernel(b0, b1, d)(xt, tabp)
    return outp[:, :, :d]
